# baseline (speedup 1.0000x reference)
import functools

import jax
import jax.numpy as jnp
from jax import lax
from jax.experimental import pallas as pl
from jax.experimental.pallas import tpu as pltpu
from jax.experimental.pallas import tpu_sc as plsc

_D = 128
_E = 16
_LANES = 16
_TILE = 128
_NW = 32


def _sc_router_body(x_hbm, ws_hbm, o_hbm, ws_v, xb_v, ob_v):
    nrows = x_hbm.shape[0] // _D
    strip = nrows // _NW
    wid = lax.axis_index("s") * 2 + lax.axis_index("c")
    base = wid * strip
    pltpu.sync_copy(ws_hbm, ws_v)
    lanes = jnp.arange(_LANES, dtype=jnp.int32)

    def tile_body(t, carry):
        r0 = base + t * _TILE
        pltpu.sync_copy(x_hbm.at[pl.ds(r0 * _D, _TILE * _D)], xb_v)
        for sub in range(_TILE // _LANES):
            rows = lanes + (sub * _LANES)

            rowbase = rows * _D

            def dbody(d, accs):
                xT = plsc.load_gather(xb_v, [rowbase + d])
                return tuple(
                    accs[e] + xT * ws_v[d, e] for e in range(_E)
                )

            accs = lax.fori_loop(
                0, _D, dbody,
                tuple(jnp.zeros((_LANES,), jnp.float32) for _ in range(_E)),
            )
            m = accs[0]
            for e in range(1, _E):
                m = jnp.maximum(m, accs[e])
            es = [jnp.exp(a - m) for a in accs]
            s = es[0]
            for e in range(1, _E):
                s = s + es[e]
            r = 1.0 / s
            for e in range(_E):
                plsc.store_scatter(ob_v, [rows * _E + e], es[e] * r)
        pltpu.sync_copy(ob_v, o_hbm.at[pl.ds(r0 * _E, _TILE * _E)])
        return carry

    lax.fori_loop(0, strip // _TILE, tile_body, 0)


def _sc_router(x2, wsplat):
    nrows = x2.shape[0] // _D
    mesh = plsc.VectorSubcoreMesh(core_axis_name="c", subcore_axis_name="s")
    f = pl.kernel(
        _sc_router_body,
        mesh=mesh,
        out_type=jax.ShapeDtypeStruct((nrows * _E,), jnp.float32),
        compiler_params=pltpu.CompilerParams(needs_layout_passes=False, use_tc_tiling_on_sc=False),
        scratch_types=[
            pltpu.VMEM((_D, _E, _LANES), jnp.float32),
            pltpu.VMEM((_TILE * _D,), jnp.float32),
            pltpu.VMEM((_TILE * _E,), jnp.float32),
        ],
    )
    return f(x2, wsplat)


def kernel(x, expert_embeddings):
    B, S, H, D = x.shape
    E = expert_embeddings.shape[1]
    R = B * S * H
    x2 = x.reshape(R * D)
    wsplat = jnp.broadcast_to(
        expert_embeddings.reshape(D, E, 1), (D, E, _LANES)
    )
    out = _sc_router(x2, wsplat)
    return out.reshape(B, S, H, E)
